# baseline (speedup 1.0000x reference)
import jax
import jax.numpy as jnp
from jax.experimental import pallas as pl
from jax.experimental.pallas import tpu as pltpu

_N = 10000
_C = 128
_ROWS = 50
_BLOCK = 400


def _body(x_ref, o_ref):
    x = x_ref[...]
    b = x.shape[0]
    o_ref[...] = jnp.zeros(o_ref.shape, o_ref.dtype)
    o_ref[:, 0:1, :] = x[:, :_C].reshape(b, 1, _C)
    o_ref[:, 25:26, :] = x[:, _C:].reshape(b, 1, _C)


def kernel(atom_embeddings):
    grid = _N // _BLOCK
    return pl.pallas_call(
        _body,
        grid=(grid,),
        in_specs=[pl.BlockSpec((_BLOCK, 2 * _C), lambda i: (i, 0))],
        out_specs=pl.BlockSpec((_BLOCK, _ROWS, _C), lambda i: (i, 0, 0)),
        out_shape=jax.ShapeDtypeStruct((_N, _ROWS, _C), atom_embeddings.dtype),
        compiler_params=pltpu.CompilerParams(
            dimension_semantics=("parallel",),
        ),
    )(atom_embeddings)
